# baseline (speedup 1.0000x reference)
import functools

import jax
import jax.numpy as jnp
from jax import lax
from jax.experimental import pallas as pl
from jax.experimental.pallas import tpu as pltpu
from jax.experimental.pallas import tpu_sc as plsc

E = 64
B = 4096
N = 1024
LANES = 128

_info = plsc.get_sparse_core_info()
_NC, _NS = _info.num_cores, _info.num_subcores
_NW = _NC * _NS
_BPW = B // _NW
_NPW = N // _NW
_BURST = 8

_sc_mesh = plsc.VectorSubcoreMesh(core_axis_name="c", subcore_axis_name="s")


def _gather_ids(tableT_hbm, idx_v, rows_v, slab_v, sem, n_ids):
    e16 = lax.iota(jnp.int32, 16)

    def chunk_body(c, _):
        cbase = c * 16
        idxvec = idx_v[pl.ds(cbase, 16)]
        for h in range(16 // _BURST):
            copies = []
            for j in range(_BURST):
                idx = idxvec[h * _BURST + j]
                start = pl.multiple_of((idx >> 7) << 7, LANES)
                copies.append(pltpu.async_copy(
                    tableT_hbm.at[:, pl.ds(start, LANES)],
                    slab_v.at[j],
                    sem))
            for cp in copies:
                cp.wait()
            for j in range(_BURST):
                idx = idxvec[h * _BURST + j]
                lane = jnp.full((16,), idx & 127, jnp.int32)
                pos = cbase + h * _BURST + j
                for k in range(E // 16):
                    vals = plsc.load_gather(slab_v.at[j], [e16 + k * 16, lane])
                    rows_v[pos, pl.ds(k * 16, 16)] = vals
        return 0
    lax.fori_loop(0, n_ids // 16, chunk_body, 0, unroll=False)


def _gather_sorted(tableT_hbm, sid_v, ord_v, rows_v, out_hbm, slab_v, sem,
                   n_ids):
    e16 = lax.iota(jnp.int32, 16)

    def chunk_body(c, col_prev):
        cbase = c * 16
        idxvec = sid_v[pl.ds(cbase, 16)]
        for j in range(16):
            idx = idxvec[j]
            col = idx >> 7
            @pl.when(col != col_prev)
            def _():
                start = pl.multiple_of((idx >> 7) << 7, LANES)
                pltpu.sync_copy(tableT_hbm.at[:, pl.ds(start, LANES)],
                                slab_v)
            lane = jnp.full((16,), idx & 127, jnp.int32)
            for k in range(E // 16):
                vals = plsc.load_gather(slab_v, [e16 + k * 16, lane])
                rows_v[cbase + j, pl.ds(k * 16, 16)] = vals
            col_prev = col
        return col_prev

    lax.fori_loop(0, n_ids // 16, chunk_body, jnp.int32(-1), unroll=False)

    def scatter_body(c, _):
        cbase = c * 16
        posvec = ord_v[pl.ds(cbase, 16)]
        copies = []
        for j in range(16):
            pos = posvec[j]
            copies.append(pltpu.async_copy(
                rows_v.at[pl.ds(cbase + j, 1), :],
                out_hbm.at[pl.ds(pos, 1), :],
                sem))
        for cp in copies:
            cp.wait()
        return 0
    lax.fori_loop(0, n_ids // 16, scatter_body, 0, unroll=False)


@functools.partial(
    pl.kernel,
    mesh=_sc_mesh,
    compiler_params=pltpu.CompilerParams(needs_layout_passes=False),
    out_type=[
        jax.ShapeDtypeStruct((B, E), jnp.float32),
        jax.ShapeDtypeStruct((B, E), jnp.float32),
        jax.ShapeDtypeStruct((N, E), jnp.float32),
        jax.ShapeDtypeStruct((N, E), jnp.float32),
    ],
    scratch_types=[
        pltpu.VMEM((_BPW,), jnp.int32),
        pltpu.VMEM((_BPW,), jnp.int32),
        pltpu.VMEM((_BPW,), jnp.int32),
        pltpu.VMEM((_BPW,), jnp.int32),
        pltpu.VMEM((_NPW,), jnp.int32),
        pltpu.VMEM((_BPW, E), jnp.float32),
        pltpu.VMEM((_BPW, E), jnp.float32),
        pltpu.VMEM((_NPW, E), jnp.float32),
        pltpu.VMEM((_NPW, E), jnp.float32),
        pltpu.VMEM((_BURST, E, LANES), jnp.float32),
        pltpu.SemaphoreType.DMA,
    ],
)
def _sc_gather(suids_hbm, uord_hbm, siids_hbm, iord_hbm, oids_hbm,
               UIt_hbm, LIt_hbm, IUt_hbm, ILt_hbm,
               ue_out, se_out, iu_out, il_out,
               suid_v, uord_v, siid_v, iord_v, oid_v,
               ue_v, se_v, iu_v, il_v, slab_v, sem):
    wid = lax.axis_index("s") * _NC + lax.axis_index("c")
    bbase = wid * _BPW
    nbase = wid * _NPW

    pltpu.sync_copy(suids_hbm.at[pl.ds(bbase, _BPW)], suid_v)
    pltpu.sync_copy(uord_hbm.at[pl.ds(bbase, _BPW)], uord_v)
    pltpu.sync_copy(siids_hbm.at[pl.ds(bbase, _BPW)], siid_v)
    pltpu.sync_copy(iord_hbm.at[pl.ds(bbase, _BPW)], iord_v)
    pltpu.sync_copy(oids_hbm.at[pl.ds(nbase, _NPW)], oid_v)

    _gather_sorted(UIt_hbm, suid_v, uord_v, ue_v, ue_out, slab_v.at[0],
                   sem, _BPW)
    _gather_sorted(LIt_hbm, siid_v, iord_v, se_v, se_out, slab_v.at[0],
                   sem, _BPW)
    _gather_ids(IUt_hbm, oid_v, iu_v, slab_v, sem, _NPW)
    _gather_ids(ILt_hbm, oid_v, il_v, slab_v, sem, _NPW)

    pltpu.sync_copy(iu_v, iu_out.at[pl.ds(nbase, _NPW)])
    pltpu.sync_copy(il_v, il_out.at[pl.ds(nbase, _NPW)])


_BM = 1024


def _mm_body(ue_ref, se_ref, iu_ref, il_ref, out_ref):
    mf = lax.dot_general(ue_ref[...], iu_ref[...], (((1,), (1,)), ((), ())),
                         precision=lax.Precision.HIGHEST,
                         preferred_element_type=jnp.float32)
    fmc = lax.dot_general(se_ref[...], il_ref[...], (((1,), (1,)), ((), ())),
                          precision=lax.Precision.HIGHEST,
                          preferred_element_type=jnp.float32)
    out_ref[...] = mf + fmc


_matmul = pl.pallas_call(
    _mm_body,
    grid=(B // _BM,),
    in_specs=[
        pl.BlockSpec((_BM, E), lambda i: (i, 0)),
        pl.BlockSpec((_BM, E), lambda i: (i, 0)),
        pl.BlockSpec((N, E), lambda i: (0, 0)),
        pl.BlockSpec((N, E), lambda i: (0, 0)),
    ],
    out_specs=pl.BlockSpec((_BM, N), lambda i: (i, 0)),
    out_shape=jax.ShapeDtypeStruct((B, N), jnp.float32),
)


def kernel(in_uids, in_iids, out_iids, UI, IU, LI, IL):
    uids = in_uids.astype(jnp.int32)
    iids = in_iids.astype(jnp.int32)
    oids = out_iids.astype(jnp.int32)
    uord = jnp.argsort(uids).astype(jnp.int32)
    suids = jnp.take(uids, uord)
    iord = jnp.argsort(iids).astype(jnp.int32)
    siids = jnp.take(iids, iord)
    ue, se, iu, il = _sc_gather(suids, uord, siids, iord, oids,
                                UI.T, LI.T, IU.T, IL.T)
    return _matmul(ue, se, iu, il)


# speedup vs baseline: 1.3252x; 1.3252x over previous
import functools

import jax
import jax.numpy as jnp
from jax import lax
from jax.experimental import pallas as pl
from jax.experimental.pallas import tpu as pltpu
from jax.experimental.pallas import tpu_sc as plsc

E = 64
B = 4096
N = 1024
LANES = 128

_info = plsc.get_sparse_core_info()
_NC, _NS = _info.num_cores, _info.num_subcores
_NW = _NC * _NS
_BPW = B // _NW
_NPW = N // _NW
_BURST = 8

_sc_mesh = plsc.VectorSubcoreMesh(core_axis_name="c", subcore_axis_name="s")


def _gather_ids(tableT_hbm, idx_v, rows_v, slab_v, sem, n_ids):
    e16 = lax.iota(jnp.int32, 16)

    def chunk_body(c, _):
        cbase = c * 16
        idxvec = idx_v[pl.ds(cbase, 16)]
        for h in range(16 // _BURST):
            copies = []
            for j in range(_BURST):
                idx = idxvec[h * _BURST + j]
                start = pl.multiple_of((idx >> 7) << 7, LANES)
                copies.append(pltpu.async_copy(
                    tableT_hbm.at[:, pl.ds(start, LANES)],
                    slab_v.at[j],
                    sem))
            for cp in copies:
                cp.wait()
            for j in range(_BURST):
                idx = idxvec[h * _BURST + j]
                lane = jnp.full((16,), idx & 127, jnp.int32)
                pos = cbase + h * _BURST + j
                for k in range(E // 16):
                    vals = plsc.load_gather(slab_v.at[j], [e16 + k * 16, lane])
                    rows_v[pos, pl.ds(k * 16, 16)] = vals
        return 0
    lax.fori_loop(0, n_ids // 16, chunk_body, 0, unroll=False)


def _gather_sorted(tableT_hbm, sid_v, ord_v, rows_v, out_hbm, slab_v, sem,
                   n_ids):
    e16 = lax.iota(jnp.int32, 16)

    def chunk_body(c, col_prev):
        cbase = c * 16
        idxvec = sid_v[pl.ds(cbase, 16)]
        for j in range(16):
            idx = idxvec[j]
            col = idx >> 7
            @pl.when(col != col_prev)
            def _():
                start = pl.multiple_of((idx >> 7) << 7, LANES)
                pltpu.sync_copy(tableT_hbm.at[:, pl.ds(start, LANES)],
                                slab_v)
            lane = jnp.full((16,), idx & 127, jnp.int32)
            for k in range(E // 16):
                vals = plsc.load_gather(slab_v, [e16 + k * 16, lane])
                rows_v[cbase + j, pl.ds(k * 16, 16)] = vals
            col_prev = col
        return col_prev

    lax.fori_loop(0, n_ids // 16, chunk_body, jnp.int32(-1), unroll=False)

    def scatter_body(c, _):
        cbase = c * 16
        posvec = ord_v[pl.ds(cbase, 16)]
        copies = []
        for j in range(16):
            pos = posvec[j]
            copies.append(pltpu.async_copy(
                rows_v.at[pl.ds(cbase + j, 1), :],
                out_hbm.at[pl.ds(pos, 1), :],
                sem))
        for cp in copies:
            cp.wait()
        return 0
    lax.fori_loop(0, n_ids // 16, scatter_body, 0, unroll=False)


@functools.partial(
    pl.kernel,
    mesh=_sc_mesh,
    compiler_params=pltpu.CompilerParams(needs_layout_passes=False),
    out_type=[
        jax.ShapeDtypeStruct((B, E), jnp.float32),
        jax.ShapeDtypeStruct((B, E), jnp.float32),
        jax.ShapeDtypeStruct((N, E), jnp.float32),
        jax.ShapeDtypeStruct((N, E), jnp.float32),
    ],
    scratch_types=[
        pltpu.VMEM((_BPW,), jnp.int32),
        pltpu.VMEM((_BPW,), jnp.int32),
        pltpu.VMEM((_BPW,), jnp.int32),
        pltpu.VMEM((_NPW,), jnp.int32),
        pltpu.VMEM((_BPW, E), jnp.float32),
        pltpu.VMEM((_BPW, E), jnp.float32),
        pltpu.VMEM((_NPW, E), jnp.float32),
        pltpu.VMEM((_NPW, E), jnp.float32),
        pltpu.VMEM((_BURST, E, LANES), jnp.float32),
        pltpu.SemaphoreType.DMA,
    ],
)
def _sc_gather(suids_hbm, uord_hbm, iids_hbm, oids_hbm,
               UIt_hbm, LIt_hbm, IUt_hbm, ILt_hbm,
               ue_out, se_out, iu_out, il_out,
               suid_v, uord_v, iid_v, oid_v,
               ue_v, se_v, iu_v, il_v, slab_v, sem):
    wid = lax.axis_index("s") * _NC + lax.axis_index("c")
    bbase = wid * _BPW
    nbase = wid * _NPW

    pltpu.sync_copy(suids_hbm.at[pl.ds(bbase, _BPW)], suid_v)
    pltpu.sync_copy(uord_hbm.at[pl.ds(bbase, _BPW)], uord_v)
    pltpu.sync_copy(iids_hbm.at[pl.ds(bbase, _BPW)], iid_v)
    pltpu.sync_copy(oids_hbm.at[pl.ds(nbase, _NPW)], oid_v)

    _gather_sorted(UIt_hbm, suid_v, uord_v, ue_v, ue_out, slab_v.at[0],
                   sem, _BPW)
    _gather_ids(LIt_hbm, iid_v, se_v, slab_v, sem, _BPW)
    _gather_ids(IUt_hbm, oid_v, iu_v, slab_v, sem, _NPW)
    _gather_ids(ILt_hbm, oid_v, il_v, slab_v, sem, _NPW)

    pltpu.sync_copy(se_v, se_out.at[pl.ds(bbase, _BPW)])
    pltpu.sync_copy(iu_v, iu_out.at[pl.ds(nbase, _NPW)])
    pltpu.sync_copy(il_v, il_out.at[pl.ds(nbase, _NPW)])


_BM = 1024


def _mm_body(ue_ref, se_ref, iu_ref, il_ref, out_ref):
    mf = lax.dot_general(ue_ref[...], iu_ref[...], (((1,), (1,)), ((), ())),
                         precision=lax.Precision.HIGHEST,
                         preferred_element_type=jnp.float32)
    fmc = lax.dot_general(se_ref[...], il_ref[...], (((1,), (1,)), ((), ())),
                          precision=lax.Precision.HIGHEST,
                          preferred_element_type=jnp.float32)
    out_ref[...] = mf + fmc


_matmul = pl.pallas_call(
    _mm_body,
    grid=(B // _BM,),
    in_specs=[
        pl.BlockSpec((_BM, E), lambda i: (i, 0)),
        pl.BlockSpec((_BM, E), lambda i: (i, 0)),
        pl.BlockSpec((N, E), lambda i: (0, 0)),
        pl.BlockSpec((N, E), lambda i: (0, 0)),
    ],
    out_specs=pl.BlockSpec((_BM, N), lambda i: (i, 0)),
    out_shape=jax.ShapeDtypeStruct((B, N), jnp.float32),
)


def kernel(in_uids, in_iids, out_iids, UI, IU, LI, IL):
    uids = in_uids.astype(jnp.int32)
    iids = in_iids.astype(jnp.int32)
    oids = out_iids.astype(jnp.int32)
    uord = jnp.argsort(uids).astype(jnp.int32)
    suids = jnp.take(uids, uord)
    ue, se, iu, il = _sc_gather(suids, uord, iids, oids,
                                UI.T, LI.T, IU.T, IL.T)
    return _matmul(ue, se, iu, il)


# speedup vs baseline: 1.5526x; 1.1716x over previous
import functools

import jax
import jax.numpy as jnp
from jax import lax
from jax.experimental import pallas as pl
from jax.experimental.pallas import tpu as pltpu
from jax.experimental.pallas import tpu_sc as plsc

E = 64
B = 4096
N = 1024
LANES = 128

_info = plsc.get_sparse_core_info()
_NC, _NS = _info.num_cores, _info.num_subcores
_NW = _NC * _NS
_BPW = B // _NW
_NPW = N // _NW
_BURST = 8

_sc_mesh = plsc.VectorSubcoreMesh(core_axis_name="c", subcore_axis_name="s")


def _gather_ids(tableT_hbm, idx_v, rows_v, slab_v, sem, n_ids):
    e16 = lax.iota(jnp.int32, 16)

    def chunk_body(c, _):
        cbase = c * 16
        idxvec = idx_v[pl.ds(cbase, 16)]
        for h in range(16 // _BURST):
            copies = []
            for j in range(_BURST):
                idx = idxvec[h * _BURST + j]
                start = pl.multiple_of((idx >> 7) << 7, LANES)
                copies.append(pltpu.async_copy(
                    tableT_hbm.at[:, pl.ds(start, LANES)],
                    slab_v.at[j],
                    sem))
            for cp in copies:
                cp.wait()
            for j in range(_BURST):
                idx = idxvec[h * _BURST + j]
                lane = jnp.full((16,), idx & 127, jnp.int32)
                pos = cbase + h * _BURST + j
                for k in range(E // 16):
                    vals = plsc.load_gather(slab_v.at[j], [e16 + k * 16, lane])
                    rows_v[pos, pl.ds(k * 16, 16)] = vals
        return 0
    lax.fori_loop(0, n_ids // 16, chunk_body, 0, unroll=False)


def _gather_sorted(tableT_hbm, sid_v, ord_v, rows_v, out_hbm, slab_v, sem,
                   n_ids):
    e16 = lax.iota(jnp.int32, 16)

    def chunk_body(c, col_prev):
        cbase = c * 16
        idxvec = sid_v[pl.ds(cbase, 16)]
        for j in range(16):
            idx = idxvec[j]
            col = idx >> 7
            @pl.when(col != col_prev)
            def _():
                start = pl.multiple_of((idx >> 7) << 7, LANES)
                pltpu.sync_copy(tableT_hbm.at[:, pl.ds(start, LANES)],
                                slab_v)
            lane = jnp.full((16,), idx & 127, jnp.int32)
            for k in range(E // 16):
                vals = plsc.load_gather(slab_v, [e16 + k * 16, lane])
                rows_v[cbase + j, pl.ds(k * 16, 16)] = vals
            col_prev = col
        return col_prev

    lax.fori_loop(0, n_ids // 16, chunk_body, jnp.int32(-1), unroll=False)

    def scatter_body(c, _):
        cbase = c * 16
        posvec = ord_v[pl.ds(cbase, 16)]
        copies = []
        for j in range(16):
            pos = posvec[j]
            copies.append(pltpu.async_copy(
                rows_v.at[pl.ds(cbase + j, 1), :],
                out_hbm.at[pl.ds(pos, 1), :],
                sem))
        for cp in copies:
            cp.wait()
        return 0
    lax.fori_loop(0, n_ids // 16, scatter_body, 0, unroll=False)


@functools.partial(
    pl.kernel,
    mesh=_sc_mesh,
    compiler_params=pltpu.CompilerParams(needs_layout_passes=False),
    out_type=[
        jax.ShapeDtypeStruct((B, E), jnp.float32),
        jax.ShapeDtypeStruct((B, E), jnp.float32),
        jax.ShapeDtypeStruct((N, E), jnp.float32),
        jax.ShapeDtypeStruct((N, E), jnp.float32),
    ],
    scratch_types=[
        pltpu.VMEM((_BPW,), jnp.int32),
        pltpu.VMEM((_BPW,), jnp.int32),
        pltpu.VMEM((_BPW,), jnp.int32),
        pltpu.VMEM((_NPW,), jnp.int32),
        pltpu.VMEM((_BPW, E), jnp.float32),
        pltpu.VMEM((_BPW, E), jnp.float32),
        pltpu.VMEM((_NPW, E), jnp.float32),
        pltpu.VMEM((_NPW, E), jnp.float32),
        pltpu.VMEM((_BURST, E, LANES), jnp.float32),
        pltpu.SemaphoreType.DMA,
    ],
)
def _sc_gather(suids_hbm, uord_hbm, iids_hbm, oids_hbm,
               UIt_hbm, LIt_hbm, IUt_hbm, ILt_hbm,
               ue_out, se_out, iu_out, il_out,
               suid_v, uord_v, iid_v, oid_v,
               ue_v, se_v, iu_v, il_v, slab_v, sem):
    wid = lax.axis_index("s") * _NC + lax.axis_index("c")
    bbase = wid * _BPW
    nbase = wid * _NPW

    pltpu.sync_copy(suids_hbm.at[pl.ds(bbase, _BPW)], suid_v)
    pltpu.sync_copy(uord_hbm.at[pl.ds(bbase, _BPW)], uord_v)
    pltpu.sync_copy(iids_hbm.at[pl.ds(bbase, _BPW)], iid_v)
    pltpu.sync_copy(oids_hbm.at[pl.ds(nbase, _NPW)], oid_v)

    _gather_sorted(UIt_hbm, suid_v, uord_v, ue_v, ue_out, slab_v.at[0],
                   sem, _BPW)
    _gather_ids(LIt_hbm, iid_v, se_v, slab_v, sem, _BPW)
    _gather_ids(IUt_hbm, oid_v, iu_v, slab_v, sem, _NPW)
    _gather_ids(ILt_hbm, oid_v, il_v, slab_v, sem, _NPW)

    pltpu.sync_copy(se_v, se_out.at[pl.ds(bbase, _BPW)])
    pltpu.sync_copy(iu_v, iu_out.at[pl.ds(nbase, _NPW)])
    pltpu.sync_copy(il_v, il_out.at[pl.ds(nbase, _NPW)])


_BM = 1024


def _mm_body(ue_ref, se_ref, iu_ref, il_ref, out_ref):
    mf = lax.dot_general(ue_ref[...], iu_ref[...], (((1,), (1,)), ((), ())),
                         preferred_element_type=jnp.float32)
    fmc = lax.dot_general(se_ref[...], il_ref[...], (((1,), (1,)), ((), ())),
                           preferred_element_type=jnp.float32)
    out_ref[...] = mf + fmc


_matmul = pl.pallas_call(
    _mm_body,
    grid=(B // _BM,),
    in_specs=[
        pl.BlockSpec((_BM, E), lambda i: (i, 0)),
        pl.BlockSpec((_BM, E), lambda i: (i, 0)),
        pl.BlockSpec((N, E), lambda i: (0, 0)),
        pl.BlockSpec((N, E), lambda i: (0, 0)),
    ],
    out_specs=pl.BlockSpec((_BM, N), lambda i: (i, 0)),
    out_shape=jax.ShapeDtypeStruct((B, N), jnp.float32),
)


def kernel(in_uids, in_iids, out_iids, UI, IU, LI, IL):
    uids = in_uids.astype(jnp.int32)
    iids = in_iids.astype(jnp.int32)
    oids = out_iids.astype(jnp.int32)
    suids, uord = lax.sort((uids, lax.iota(jnp.int32, B)), num_keys=1)
    ue, se, iu, il = _sc_gather(suids, uord, iids, oids,
                                UI.T, LI.T, IU.T, IL.T)
    return _matmul(ue, se, iu, il)


# speedup vs baseline: 1.7158x; 1.1051x over previous
import functools

import jax
import jax.numpy as jnp
from jax import lax
from jax.experimental import pallas as pl
from jax.experimental.pallas import tpu as pltpu
from jax.experimental.pallas import tpu_sc as plsc

E = 64
B = 4096
N = 1024
LANES = 128

_info = plsc.get_sparse_core_info()
_NC, _NS = _info.num_cores, _info.num_subcores
_NW = _NC * _NS
_BPW = B // _NW
_NPW = N // _NW
_BURST = 8

_sc_mesh = plsc.VectorSubcoreMesh(core_axis_name="c", subcore_axis_name="s")


def _gather_ids(tableT_hbm, idx_v, rows_v, slab_v, sem, n_ids):
    e16 = lax.iota(jnp.int32, 16)

    def chunk_body(c, _):
        cbase = c * 16
        idxvec = idx_v[pl.ds(cbase, 16)]

        def fire(b):
            copies = []
            for j in range(4):
                idx = idxvec[b * 4 + j]
                start = pl.multiple_of((idx >> 7) << 7, LANES)
                copies.append(pltpu.async_copy(
                    tableT_hbm.at[:, pl.ds(start, LANES)],
                    slab_v.at[(b % 2) * 4 + j],
                    sem))
            return copies

        def extract(b):
            for j in range(4):
                idx = idxvec[b * 4 + j]
                lane = jnp.full((16,), idx & 127, jnp.int32)
                pos = cbase + b * 4 + j
                for k in range(E // 16):
                    vals = plsc.load_gather(slab_v.at[(b % 2) * 4 + j],
                                            [e16 + k * 16, lane])
                    rows_v[pos, pl.ds(k * 16, 16)] = vals

        pend = {0: fire(0), 1: fire(1)}
        for b in range(4):
            for cp in pend[b]:
                cp.wait()
            extract(b)
            if b + 2 < 4:
                pend[b + 2] = fire(b + 2)
        return 0
    lax.fori_loop(0, n_ids // 16, chunk_body, 0, unroll=False)


def _gather_sorted(tableT_hbm, sid_v, ord_v, rows_v, out_hbm, slab_v, sem,
                   n_ids):
    e16 = lax.iota(jnp.int32, 16)

    def chunk_body(c, col_prev):
        cbase = c * 16
        idxvec = sid_v[pl.ds(cbase, 16)]
        for j in range(16):
            idx = idxvec[j]
            col = idx >> 7
            @pl.when(col != col_prev)
            def _():
                start = pl.multiple_of((idx >> 7) << 7, LANES)
                pltpu.sync_copy(tableT_hbm.at[:, pl.ds(start, LANES)],
                                slab_v)
            lane = jnp.full((16,), idx & 127, jnp.int32)
            for k in range(E // 16):
                vals = plsc.load_gather(slab_v, [e16 + k * 16, lane])
                rows_v[cbase + j, pl.ds(k * 16, 16)] = vals
            col_prev = col
        return col_prev

    lax.fori_loop(0, n_ids // 16, chunk_body, jnp.int32(-1), unroll=False)

    def scatter_body(c, _):
        cbase = c * 16
        posvec = ord_v[pl.ds(cbase, 16)]
        copies = []
        for j in range(16):
            pos = posvec[j]
            copies.append(pltpu.async_copy(
                rows_v.at[pl.ds(cbase + j, 1), :],
                out_hbm.at[pl.ds(pos, 1), :],
                sem))
        for cp in copies:
            cp.wait()
        return 0
    lax.fori_loop(0, n_ids // 16, scatter_body, 0, unroll=False)


@functools.partial(
    pl.kernel,
    mesh=_sc_mesh,
    compiler_params=pltpu.CompilerParams(needs_layout_passes=False),
    out_type=[
        jax.ShapeDtypeStruct((B, E), jnp.float32),
        jax.ShapeDtypeStruct((B, E), jnp.float32),
        jax.ShapeDtypeStruct((N, E), jnp.float32),
        jax.ShapeDtypeStruct((N, E), jnp.float32),
    ],
    scratch_types=[
        pltpu.VMEM((_BPW,), jnp.int32),
        pltpu.VMEM((_BPW,), jnp.int32),
        pltpu.VMEM((_BPW,), jnp.int32),
        pltpu.VMEM((_NPW,), jnp.int32),
        pltpu.VMEM((_BPW, E), jnp.float32),
        pltpu.VMEM((_BPW, E), jnp.float32),
        pltpu.VMEM((_NPW, E), jnp.float32),
        pltpu.VMEM((_NPW, E), jnp.float32),
        pltpu.VMEM((_BURST, E, LANES), jnp.float32),
        pltpu.SemaphoreType.DMA,
    ],
)
def _sc_gather(suids_hbm, uord_hbm, iids_hbm, oids_hbm,
               UIt_hbm, LIt_hbm, IUt_hbm, ILt_hbm,
               ue_out, se_out, iu_out, il_out,
               suid_v, uord_v, iid_v, oid_v,
               ue_v, se_v, iu_v, il_v, slab_v, sem):
    wid = lax.axis_index("s") * _NC + lax.axis_index("c")
    bbase = wid * _BPW
    nbase = wid * _NPW

    pltpu.sync_copy(suids_hbm.at[pl.ds(bbase, _BPW)], suid_v)
    pltpu.sync_copy(uord_hbm.at[pl.ds(bbase, _BPW)], uord_v)
    pltpu.sync_copy(iids_hbm.at[pl.ds(bbase, _BPW)], iid_v)
    pltpu.sync_copy(oids_hbm.at[pl.ds(nbase, _NPW)], oid_v)

    _gather_sorted(UIt_hbm, suid_v, uord_v, ue_v, ue_out, slab_v.at[0],
                   sem, _BPW)
    _gather_ids(LIt_hbm, iid_v, se_v, slab_v, sem, _BPW)
    _gather_ids(IUt_hbm, oid_v, iu_v, slab_v, sem, _NPW)
    _gather_ids(ILt_hbm, oid_v, il_v, slab_v, sem, _NPW)

    pltpu.sync_copy(se_v, se_out.at[pl.ds(bbase, _BPW)])
    pltpu.sync_copy(iu_v, iu_out.at[pl.ds(nbase, _NPW)])
    pltpu.sync_copy(il_v, il_out.at[pl.ds(nbase, _NPW)])


_BM = 1024


def _mm_body(ue_ref, se_ref, iu_ref, il_ref, out_ref):
    mf = lax.dot_general(ue_ref[...], iu_ref[...], (((1,), (1,)), ((), ())),
                         preferred_element_type=jnp.float32)
    fmc = lax.dot_general(se_ref[...], il_ref[...], (((1,), (1,)), ((), ())),
                           preferred_element_type=jnp.float32)
    out_ref[...] = mf + fmc


_matmul = pl.pallas_call(
    _mm_body,
    grid=(B // _BM,),
    in_specs=[
        pl.BlockSpec((_BM, E), lambda i: (i, 0)),
        pl.BlockSpec((_BM, E), lambda i: (i, 0)),
        pl.BlockSpec((N, E), lambda i: (0, 0)),
        pl.BlockSpec((N, E), lambda i: (0, 0)),
    ],
    out_specs=pl.BlockSpec((_BM, N), lambda i: (i, 0)),
    out_shape=jax.ShapeDtypeStruct((B, N), jnp.float32),
)


def kernel(in_uids, in_iids, out_iids, UI, IU, LI, IL):
    uids = in_uids.astype(jnp.int32)
    iids = in_iids.astype(jnp.int32)
    oids = out_iids.astype(jnp.int32)
    suids, uord = lax.sort((uids, lax.iota(jnp.int32, B)), num_keys=1)
    ue, se, iu, il = _sc_gather(suids, uord, iids, oids,
                                UI.T, LI.T, IU.T, IL.T)
    return _matmul(ue, se, iu, il)


# speedup vs baseline: 1.7190x; 1.0019x over previous
import functools

import jax
import jax.numpy as jnp
from jax import lax
from jax.experimental import pallas as pl
from jax.experimental.pallas import tpu as pltpu
from jax.experimental.pallas import tpu_sc as plsc

E = 64
B = 4096
N = 1024
LANES = 128

_info = plsc.get_sparse_core_info()
_NC, _NS = _info.num_cores, _info.num_subcores
_NW = _NC * _NS
_BPW = B // _NW
_NPW = N // _NW
_NSLAB = 8

_sc_mesh = plsc.VectorSubcoreMesh(core_axis_name="c", subcore_axis_name="s")


def _gather_ids(tableT_hbm, idx_v, rows_v, slab_v, sem, n_ids):
    e16 = lax.iota(jnp.int32, 16)

    def chunk_body(c, _):
        cbase = c * 16
        idxvec = idx_v[pl.ds(cbase, 16)]

        def fire(b):
            copies = []
            for j in range(4):
                idx = idxvec[b * 4 + j]
                start = pl.multiple_of((idx >> 7) << 7, LANES)
                copies.append(pltpu.async_copy(
                    tableT_hbm.at[:, pl.ds(start, LANES)],
                    slab_v.at[(b % 2) * 4 + j],
                    sem))
            return copies

        def extract(b):
            for j in range(4):
                idx = idxvec[b * 4 + j]
                lane = jnp.full((16,), idx & 127, jnp.int32)
                pos = cbase + b * 4 + j
                for k in range(E // 16):
                    vals = plsc.load_gather(slab_v.at[(b % 2) * 4 + j],
                                            [e16 + k * 16, lane])
                    rows_v[pos, pl.ds(k * 16, 16)] = vals

        pend = {0: fire(0), 1: fire(1)}
        for b in range(4):
            for cp in pend[b]:
                cp.wait()
            extract(b)
            if b + 2 < 4:
                pend[b + 2] = fire(b + 2)
        return 0
    lax.fori_loop(0, n_ids // 16, chunk_body, 0, unroll=False)


def _gather_sorted(tableT_hbm, sid_v, ord_v, rows_v, out_hbm, slab_v, sem,
                   n_ids):
    e16 = lax.iota(jnp.int32, 16)

    def chunk_body(c, col_prev):
        cbase = c * 16
        idxvec = sid_v[pl.ds(cbase, 16)]
        for j in range(16):
            idx = idxvec[j]
            col = idx >> 7
            @pl.when(col != col_prev)
            def _():
                start = pl.multiple_of((idx >> 7) << 7, LANES)
                pltpu.sync_copy(tableT_hbm.at[:, pl.ds(start, LANES)],
                                slab_v)
            lane = jnp.full((16,), idx & 127, jnp.int32)
            for k in range(E // 16):
                vals = plsc.load_gather(slab_v, [e16 + k * 16, lane])
                rows_v[cbase + j, pl.ds(k * 16, 16)] = vals
            col_prev = col
        return col_prev

    lax.fori_loop(0, n_ids // 16, chunk_body, jnp.int32(-1), unroll=False)

    def scatter_body(c, _):
        cbase = c * 16
        posvec = ord_v[pl.ds(cbase, 16)]
        copies = []
        for j in range(16):
            pos = posvec[j]
            copies.append(pltpu.async_copy(
                rows_v.at[pl.ds(cbase + j, 1), :],
                out_hbm.at[pl.ds(pos, 1), :],
                sem))
        for cp in copies:
            cp.wait()
        return 0
    lax.fori_loop(0, n_ids // 16, scatter_body, 0, unroll=False)


@functools.partial(
    pl.kernel,
    mesh=_sc_mesh,
    compiler_params=pltpu.CompilerParams(needs_layout_passes=False),
    out_type=[
        jax.ShapeDtypeStruct((B, E), jnp.float32),
        jax.ShapeDtypeStruct((B, E), jnp.float32),
        jax.ShapeDtypeStruct((N, E), jnp.float32),
        jax.ShapeDtypeStruct((N, E), jnp.float32),
    ],
    scratch_types=[
        pltpu.VMEM((_BPW,), jnp.int32),
        pltpu.VMEM((_BPW,), jnp.int32),
        pltpu.VMEM((_BPW,), jnp.int32),
        pltpu.VMEM((_NPW,), jnp.int32),
        pltpu.VMEM((_BPW, E), jnp.float32),
        pltpu.VMEM((_BPW, E), jnp.float32),
        pltpu.VMEM((_NPW, E), jnp.float32),
        pltpu.VMEM((_NPW, E), jnp.float32),
        pltpu.VMEM((_NSLAB, E, LANES), jnp.float32),
        pltpu.SemaphoreType.DMA,
    ],
)
def _sc_gather(suids_hbm, uord_hbm, iids_hbm, oids_hbm,
               UIt_hbm, LIt_hbm, IUt_hbm, ILt_hbm,
               ue_out, se_out, iu_out, il_out,
               suid_v, uord_v, iid_v, oid_v,
               ue_v, se_v, iu_v, il_v, slab_v, sem):
    wid = lax.axis_index("s") * _NC + lax.axis_index("c")
    bbase = wid * _BPW
    nbase = wid * _NPW

    pltpu.sync_copy(suids_hbm.at[pl.ds(bbase, _BPW)], suid_v)
    pltpu.sync_copy(uord_hbm.at[pl.ds(bbase, _BPW)], uord_v)
    pltpu.sync_copy(iids_hbm.at[pl.ds(bbase, _BPW)], iid_v)
    pltpu.sync_copy(oids_hbm.at[pl.ds(nbase, _NPW)], oid_v)

    _gather_sorted(UIt_hbm, suid_v, uord_v, ue_v, ue_out, slab_v.at[0],
                   sem, _BPW)
    _gather_ids(LIt_hbm, iid_v, se_v, slab_v, sem, _BPW)
    _gather_ids(IUt_hbm, oid_v, iu_v, slab_v, sem, _NPW)
    _gather_ids(ILt_hbm, oid_v, il_v, slab_v, sem, _NPW)

    pltpu.sync_copy(se_v, se_out.at[pl.ds(bbase, _BPW)])
    pltpu.sync_copy(iu_v, iu_out.at[pl.ds(nbase, _NPW)])
    pltpu.sync_copy(il_v, il_out.at[pl.ds(nbase, _NPW)])


_BM = 1024


def _mm_body(ue_ref, se_ref, iu_ref, il_ref, out_ref):
    mf = lax.dot_general(ue_ref[...], iu_ref[...], (((1,), (1,)), ((), ())),
                         preferred_element_type=jnp.float32)
    fmc = lax.dot_general(se_ref[...], il_ref[...], (((1,), (1,)), ((), ())),
                           preferred_element_type=jnp.float32)
    out_ref[...] = mf + fmc


_matmul = pl.pallas_call(
    _mm_body,
    grid=(B // _BM,),
    in_specs=[
        pl.BlockSpec((_BM, E), lambda i: (i, 0)),
        pl.BlockSpec((_BM, E), lambda i: (i, 0)),
        pl.BlockSpec((N, E), lambda i: (0, 0)),
        pl.BlockSpec((N, E), lambda i: (0, 0)),
    ],
    out_specs=pl.BlockSpec((_BM, N), lambda i: (i, 0)),
    out_shape=jax.ShapeDtypeStruct((B, N), jnp.float32),
)


def kernel(in_uids, in_iids, out_iids, UI, IU, LI, IL):
    uids = in_uids.astype(jnp.int32)
    iids = in_iids.astype(jnp.int32)
    oids = out_iids.astype(jnp.int32)
    suids, uord = lax.sort((uids, lax.iota(jnp.int32, B)), num_keys=1)
    ue, se, iu, il = _sc_gather(suids, uord, iids, oids,
                                UI.T, LI.T, IU.T, IL.T)
    return _matmul(ue, se, iu, il)


# speedup vs baseline: 1.7571x; 1.0221x over previous
import functools

import jax
import jax.numpy as jnp
from jax import lax
from jax.experimental import pallas as pl
from jax.experimental.pallas import tpu as pltpu
from jax.experimental.pallas import tpu_sc as plsc

E = 64
B = 4096
N = 1024
LANES = 128

_info = plsc.get_sparse_core_info()
_NC, _NS = _info.num_cores, _info.num_subcores
_NW = _NC * _NS
_BPW = B // _NW
_NPW = N // _NW
_NSLAB = 8

_sc_mesh = plsc.VectorSubcoreMesh(core_axis_name="c", subcore_axis_name="s")


def _gather_ids(tableT_hbm, idx_v, rows_v, slab_v, sem, n_ids):
    e16 = lax.iota(jnp.int32, 16)
    idxvecs = [idx_v[pl.ds(c * 16, 16)] for c in range(n_ids // 16)]

    def _id(g, j):
        c, b = divmod(g, 4)
        return idxvecs[c][b * 4 + j]

    def fire(g):
        copies = []
        for j in range(4):
            idx = _id(g, j)
            start = pl.multiple_of((idx >> 7) << 7, LANES)
            copies.append(pltpu.async_copy(
                tableT_hbm.at[:, pl.ds(start, LANES)],
                slab_v.at[(g % 2) * 4 + j],
                sem))
        return copies

    def extract(g):
        for j in range(4):
            idx = _id(g, j)
            lane = jnp.full((16,), idx & 127, jnp.int32)
            pos = g * 4 + j
            for k in range(E // 16):
                vals = plsc.load_gather(slab_v.at[(g % 2) * 4 + j],
                                        [e16 + k * 16, lane])
                rows_v[pos, pl.ds(k * 16, 16)] = vals

    n_bursts = n_ids // 4
    pend = {0: fire(0), 1: fire(1)}
    for g in range(n_bursts):
        for cp in pend[g]:
            cp.wait()
        extract(g)
        if g + 2 < n_bursts:
            pend[g + 2] = fire(g + 2)


def _gather_sorted(tableT_hbm, sid_v, ord_v, rows_v, out_hbm, slab_v, sem,
                   n_ids):
    e16 = lax.iota(jnp.int32, 16)

    def chunk_body(c, col_prev):
        cbase = c * 16
        idxvec = sid_v[pl.ds(cbase, 16)]
        for j in range(16):
            idx = idxvec[j]
            col = idx >> 7
            @pl.when(col != col_prev)
            def _():
                start = pl.multiple_of((idx >> 7) << 7, LANES)
                pltpu.sync_copy(tableT_hbm.at[:, pl.ds(start, LANES)],
                                slab_v)
            lane = jnp.full((16,), idx & 127, jnp.int32)
            for k in range(E // 16):
                vals = plsc.load_gather(slab_v, [e16 + k * 16, lane])
                rows_v[cbase + j, pl.ds(k * 16, 16)] = vals
            col_prev = col
        return col_prev

    lax.fori_loop(0, n_ids // 16, chunk_body, jnp.int32(-1), unroll=False)

    def scatter_body(c, _):
        cbase = c * 16
        posvec = ord_v[pl.ds(cbase, 16)]
        copies = []
        for j in range(16):
            pos = posvec[j]
            copies.append(pltpu.async_copy(
                rows_v.at[pl.ds(cbase + j, 1), :],
                out_hbm.at[pl.ds(pos, 1), :],
                sem))
        for cp in copies:
            cp.wait()
        return 0
    lax.fori_loop(0, n_ids // 16, scatter_body, 0, unroll=False)


@functools.partial(
    pl.kernel,
    mesh=_sc_mesh,
    compiler_params=pltpu.CompilerParams(needs_layout_passes=False),
    out_type=[
        jax.ShapeDtypeStruct((B, E), jnp.float32),
        jax.ShapeDtypeStruct((B, E), jnp.float32),
        jax.ShapeDtypeStruct((N, E), jnp.float32),
        jax.ShapeDtypeStruct((N, E), jnp.float32),
    ],
    scratch_types=[
        pltpu.VMEM((_BPW,), jnp.int32),
        pltpu.VMEM((_BPW,), jnp.int32),
        pltpu.VMEM((_BPW,), jnp.int32),
        pltpu.VMEM((_NPW,), jnp.int32),
        pltpu.VMEM((_BPW, E), jnp.float32),
        pltpu.VMEM((_BPW, E), jnp.float32),
        pltpu.VMEM((_NPW, E), jnp.float32),
        pltpu.VMEM((_NPW, E), jnp.float32),
        pltpu.VMEM((_NSLAB, E, LANES), jnp.float32),
        pltpu.SemaphoreType.DMA,
    ],
)
def _sc_gather(suids_hbm, uord_hbm, iids_hbm, oids_hbm,
               UIt_hbm, LIt_hbm, IUt_hbm, ILt_hbm,
               ue_out, se_out, iu_out, il_out,
               suid_v, uord_v, iid_v, oid_v,
               ue_v, se_v, iu_v, il_v, slab_v, sem):
    wid = lax.axis_index("s") * _NC + lax.axis_index("c")
    bbase = wid * _BPW
    nbase = wid * _NPW

    pltpu.sync_copy(suids_hbm.at[pl.ds(bbase, _BPW)], suid_v)
    pltpu.sync_copy(uord_hbm.at[pl.ds(bbase, _BPW)], uord_v)
    pltpu.sync_copy(iids_hbm.at[pl.ds(bbase, _BPW)], iid_v)
    pltpu.sync_copy(oids_hbm.at[pl.ds(nbase, _NPW)], oid_v)

    _gather_sorted(UIt_hbm, suid_v, uord_v, ue_v, ue_out, slab_v.at[0],
                   sem, _BPW)
    _gather_ids(LIt_hbm, iid_v, se_v, slab_v, sem, _BPW)
    _gather_ids(IUt_hbm, oid_v, iu_v, slab_v, sem, _NPW)
    _gather_ids(ILt_hbm, oid_v, il_v, slab_v, sem, _NPW)

    pltpu.sync_copy(se_v, se_out.at[pl.ds(bbase, _BPW)])
    pltpu.sync_copy(iu_v, iu_out.at[pl.ds(nbase, _NPW)])
    pltpu.sync_copy(il_v, il_out.at[pl.ds(nbase, _NPW)])


_BM = 1024


def _mm_body(ue_ref, se_ref, iu_ref, il_ref, out_ref):
    mf = lax.dot_general(ue_ref[...], iu_ref[...], (((1,), (1,)), ((), ())),
                         preferred_element_type=jnp.float32)
    fmc = lax.dot_general(se_ref[...], il_ref[...], (((1,), (1,)), ((), ())),
                           preferred_element_type=jnp.float32)
    out_ref[...] = mf + fmc


_matmul = pl.pallas_call(
    _mm_body,
    grid=(B // _BM,),
    in_specs=[
        pl.BlockSpec((_BM, E), lambda i: (i, 0)),
        pl.BlockSpec((_BM, E), lambda i: (i, 0)),
        pl.BlockSpec((N, E), lambda i: (0, 0)),
        pl.BlockSpec((N, E), lambda i: (0, 0)),
    ],
    out_specs=pl.BlockSpec((_BM, N), lambda i: (i, 0)),
    out_shape=jax.ShapeDtypeStruct((B, N), jnp.float32),
)


def kernel(in_uids, in_iids, out_iids, UI, IU, LI, IL):
    uids = in_uids.astype(jnp.int32)
    iids = in_iids.astype(jnp.int32)
    oids = out_iids.astype(jnp.int32)
    suids, uord = lax.sort((uids, lax.iota(jnp.int32, B)), num_keys=1)
    ue, se, iu, il = _sc_gather(suids, uord, iids, oids,
                                UI.T, LI.T, IU.T, IL.T)
    return _matmul(ue, se, iu, il)


# speedup vs baseline: 1.7582x; 1.0006x over previous
import functools

import jax
import jax.numpy as jnp
from jax import lax
from jax.experimental import pallas as pl
from jax.experimental.pallas import tpu as pltpu
from jax.experimental.pallas import tpu_sc as plsc

E = 64
B = 4096
N = 1024
LANES = 128

_info = plsc.get_sparse_core_info()
_NC, _NS = _info.num_cores, _info.num_subcores
_NW = _NC * _NS
_BPW = B // _NW
_NPW = N // _NW
_NSLAB = 8

_sc_mesh = plsc.VectorSubcoreMesh(core_axis_name="c", subcore_axis_name="s")


def _gather_ids(tableT_hbm, idx_v, rows_v, slab_v, sem, n_ids):
    e16 = lax.iota(jnp.int32, 16)
    idxvecs = [idx_v[pl.ds(c * 16, 16)] for c in range(n_ids // 16)]

    def _id(g, j):
        c, b = divmod(g, 8)
        return idxvecs[c][b * 2 + j]

    def fire(g):
        copies = []
        for j in range(2):
            idx = _id(g, j)
            start = pl.multiple_of((idx >> 7) << 7, LANES)
            copies.append(pltpu.async_copy(
                tableT_hbm.at[:, pl.ds(start, LANES)],
                slab_v.at[(g % 4) * 2 + j],
                sem))
        return copies

    def extract(g):
        for j in range(2):
            idx = _id(g, j)
            lane = jnp.full((16,), idx & 127, jnp.int32)
            pos = g * 2 + j
            for k in range(E // 16):
                vals = plsc.load_gather(slab_v.at[(g % 4) * 2 + j],
                                        [e16 + k * 16, lane])
                rows_v[pos, pl.ds(k * 16, 16)] = vals

    n_bursts = n_ids // 2
    pend = {0: fire(0), 1: fire(1), 2: fire(2)}
    for g in range(n_bursts):
        for cp in pend[g]:
            cp.wait()
        extract(g)
        if g + 3 < n_bursts:
            pend[g + 3] = fire(g + 3)


def _gather_sorted(tableT_hbm, sid_v, ord_v, rows_v, out_hbm, slab_v, sem,
                   n_ids):
    e16 = lax.iota(jnp.int32, 16)

    def chunk_body(c, col_prev):
        cbase = c * 16
        idxvec = sid_v[pl.ds(cbase, 16)]
        for j in range(16):
            idx = idxvec[j]
            col = idx >> 7
            @pl.when(col != col_prev)
            def _():
                start = pl.multiple_of((idx >> 7) << 7, LANES)
                pltpu.sync_copy(tableT_hbm.at[:, pl.ds(start, LANES)],
                                slab_v)
            lane = jnp.full((16,), idx & 127, jnp.int32)
            for k in range(E // 16):
                vals = plsc.load_gather(slab_v, [e16 + k * 16, lane])
                rows_v[cbase + j, pl.ds(k * 16, 16)] = vals
            col_prev = col
        return col_prev

    lax.fori_loop(0, n_ids // 16, chunk_body, jnp.int32(-1), unroll=False)

    def scatter_body(c, _):
        cbase = c * 16
        posvec = ord_v[pl.ds(cbase, 16)]
        copies = []
        for j in range(16):
            pos = posvec[j]
            copies.append(pltpu.async_copy(
                rows_v.at[pl.ds(cbase + j, 1), :],
                out_hbm.at[pl.ds(pos, 1), :],
                sem))
        for cp in copies:
            cp.wait()
        return 0
    lax.fori_loop(0, n_ids // 16, scatter_body, 0, unroll=False)


@functools.partial(
    pl.kernel,
    mesh=_sc_mesh,
    compiler_params=pltpu.CompilerParams(needs_layout_passes=False),
    out_type=[
        jax.ShapeDtypeStruct((B, E), jnp.float32),
        jax.ShapeDtypeStruct((B, E), jnp.float32),
        jax.ShapeDtypeStruct((N, E), jnp.float32),
        jax.ShapeDtypeStruct((N, E), jnp.float32),
    ],
    scratch_types=[
        pltpu.VMEM((_BPW,), jnp.int32),
        pltpu.VMEM((_BPW,), jnp.int32),
        pltpu.VMEM((_BPW,), jnp.int32),
        pltpu.VMEM((_NPW,), jnp.int32),
        pltpu.VMEM((_BPW, E), jnp.float32),
        pltpu.VMEM((_BPW, E), jnp.float32),
        pltpu.VMEM((_NPW, E), jnp.float32),
        pltpu.VMEM((_NPW, E), jnp.float32),
        pltpu.VMEM((_NSLAB, E, LANES), jnp.float32),
        pltpu.SemaphoreType.DMA,
    ],
)
def _sc_gather(suids_hbm, uord_hbm, iids_hbm, oids_hbm,
               UIt_hbm, LIt_hbm, IUt_hbm, ILt_hbm,
               ue_out, se_out, iu_out, il_out,
               suid_v, uord_v, iid_v, oid_v,
               ue_v, se_v, iu_v, il_v, slab_v, sem):
    wid = lax.axis_index("s") * _NC + lax.axis_index("c")
    bbase = wid * _BPW
    nbase = wid * _NPW

    pltpu.sync_copy(suids_hbm.at[pl.ds(bbase, _BPW)], suid_v)
    pltpu.sync_copy(uord_hbm.at[pl.ds(bbase, _BPW)], uord_v)
    pltpu.sync_copy(iids_hbm.at[pl.ds(bbase, _BPW)], iid_v)
    pltpu.sync_copy(oids_hbm.at[pl.ds(nbase, _NPW)], oid_v)

    _gather_sorted(UIt_hbm, suid_v, uord_v, ue_v, ue_out, slab_v.at[0],
                   sem, _BPW)
    _gather_ids(LIt_hbm, iid_v, se_v, slab_v, sem, _BPW)
    _gather_ids(IUt_hbm, oid_v, iu_v, slab_v, sem, _NPW)
    _gather_ids(ILt_hbm, oid_v, il_v, slab_v, sem, _NPW)

    pltpu.sync_copy(se_v, se_out.at[pl.ds(bbase, _BPW)])
    pltpu.sync_copy(iu_v, iu_out.at[pl.ds(nbase, _NPW)])
    pltpu.sync_copy(il_v, il_out.at[pl.ds(nbase, _NPW)])


_BM = 1024


def _mm_body(ue_ref, se_ref, iu_ref, il_ref, out_ref):
    mf = lax.dot_general(ue_ref[...], iu_ref[...], (((1,), (1,)), ((), ())),
                         preferred_element_type=jnp.float32)
    fmc = lax.dot_general(se_ref[...], il_ref[...], (((1,), (1,)), ((), ())),
                           preferred_element_type=jnp.float32)
    out_ref[...] = mf + fmc


_matmul = pl.pallas_call(
    _mm_body,
    grid=(B // _BM,),
    in_specs=[
        pl.BlockSpec((_BM, E), lambda i: (i, 0)),
        pl.BlockSpec((_BM, E), lambda i: (i, 0)),
        pl.BlockSpec((N, E), lambda i: (0, 0)),
        pl.BlockSpec((N, E), lambda i: (0, 0)),
    ],
    out_specs=pl.BlockSpec((_BM, N), lambda i: (i, 0)),
    out_shape=jax.ShapeDtypeStruct((B, N), jnp.float32),
)


def kernel(in_uids, in_iids, out_iids, UI, IU, LI, IL):
    uids = in_uids.astype(jnp.int32)
    iids = in_iids.astype(jnp.int32)
    oids = out_iids.astype(jnp.int32)
    suids, uord = lax.sort((uids, lax.iota(jnp.int32, B)), num_keys=1)
    ue, se, iu, il = _sc_gather(suids, uord, iids, oids,
                                UI.T, LI.T, IU.T, IL.T)
    return _matmul(ue, se, iu, il)
